# baseline (speedup 1.0000x reference)
import jax
import jax.numpy as jnp
from jax import lax
from jax.experimental import pallas as pl
from jax.experimental.pallas import tpu as pltpu
from jax.experimental.pallas import tpu_sc as plsc

D_MODEL = 128
CHUNK = 128


def _embed_lookup(x_flat, weight, nw, n_chunks):
    mesh = plsc.VectorSubcoreMesh(core_axis_name="c", subcore_axis_name="s")
    nc = mesh.num_cores

    nbuf = 2
    gpb = 2
    rows_per_buf = gpb * CHUNK
    n_sto = n_chunks // gpb
    assert n_sto % nbuf == 0 and n_sto > 2 * nbuf

    @pl.kernel(
        out_type=jax.ShapeDtypeStruct((nw, n_sto, rows_per_buf, D_MODEL), jnp.float32),
        mesh=mesh,
        scratch_types=[
            pltpu.VMEM((n_chunks, CHUNK), jnp.int32),
            *([pltpu.VMEM((rows_per_buf, D_MODEL), jnp.float32)] * nbuf),
            *([pltpu.SemaphoreType.DMA] * (2 * nbuf)),
        ],
    )
    def k(x_hbm, w_hbm, out_hbm, idx_v, *bufs_and_sems):
        bufs = bufs_and_sems[:nbuf]
        gsem = bufs_and_sems[nbuf : 2 * nbuf]
        ssem = bufs_and_sems[2 * nbuf :]
        wid = lax.axis_index("s") * nc + lax.axis_index("c")
        pltpu.sync_copy(x_hbm.at[wid], idx_v)

        def start_gathers(b, t):
            for g in range(gpb):
                pltpu.async_copy(
                    w_hbm.at[idx_v.at[t * gpb + g]],
                    bufs[b].at[pl.ds(g * CHUNK, CHUNK)],
                    gsem[b],
                )

        def wait_gathers(b):
            for g in range(gpb):
                pltpu.make_async_copy(
                    w_hbm.at[idx_v.at[0]], bufs[b].at[pl.ds(g * CHUNK, CHUNK)], gsem[b]
                ).wait()

        def wait_store(b, t):
            pltpu.make_async_copy(bufs[b], out_hbm.at[wid, t], ssem[b]).wait()

        for b in range(nbuf):
            start_gathers(b, b)

        @pl.loop(0, n_sto - nbuf, step=nbuf)
        def _(j):
            for b in range(nbuf):
                t = j + b
                wait_gathers(b)
                pltpu.async_copy(bufs[b], out_hbm.at[wid, t], ssem[b])
                wait_store(b, t)
                start_gathers(b, t + nbuf)

        for b in range(nbuf):
            t = n_sto - nbuf + b
            wait_gathers(b)
            pltpu.async_copy(bufs[b], out_hbm.at[wid, t], ssem[b])
        for b in range(nbuf):
            wait_store(b, n_sto - nbuf + b)

    return k(x_flat, weight)


@jax.jit
def kernel(x, weight):
    batch, hist = x.shape
    total = batch * hist
    info = plsc.get_sparse_core_info()
    nw = info.num_cores * info.num_subcores
    n_chunks = total // (nw * CHUNK)
    x_flat = x.astype(jnp.int32).reshape(nw, n_chunks, CHUNK)
    out = _embed_lookup(x_flat, weight, nw, n_chunks)
    return out.reshape(batch, hist, weight.shape[1])
